# baseline (speedup 1.0000x reference)
import functools

import jax
import jax.numpy as jnp
from jax import lax
from jax.experimental import pallas as pl
from jax.experimental.pallas import tpu as pltpu
from jax.experimental.pallas import tpu_sc as plsc

_N = 10000
_NPAD = 10240
_E = 320000
_CHUNK = 128
_NC = 2
_NS = 16
_NW = _NC * _NS
_NT = 80
_NCHUNKS = _NW * _NT
_EP = _NCHUNKS * _CHUNK
_RPT = _NPAD // _NS


def _sc_mesh():
    return plsc.VectorSubcoreMesh(core_axis_name="c", subcore_axis_name="s")


def _segsum(xs, eidx, feat):
    @functools.partial(
        pl.kernel,
        out_type=jax.ShapeDtypeStruct((_NC, _NPAD, feat), jnp.float32),
        mesh=_sc_mesh(),
        scratch_types=[
            pltpu.VMEM((2, 16, _CHUNK), jnp.int32),
            pltpu.VMEM((2, _CHUNK, feat), jnp.float32),
            pltpu.VMEM_SHARED((_NPAD, feat), jnp.float32),
            pltpu.SemaphoreType.DMA,
            pltpu.SemaphoreType.DMA,
            pltpu.SemaphoreType.DMA,
            pltpu.SemaphoreType.DMA,
        ],
    )
    def k(xs_hbm, eidx_hbm, zeros_hbm, out_hbm, idx_all, rows_v, acc,
          g0, g1, s0, s1):
        gsems = (g0, g1)
        ssems = (s0, s1)
        c = lax.axis_index("c")
        s = lax.axis_index("s")
        wid = s * _NC + c
        base = s * _RPT
        pltpu.sync_copy(zeros_hbm.at[pl.ds(base, _RPT)], acc.at[pl.ds(base, _RPT)])
        plsc.subcore_barrier()

        @pl.loop(0, _NT // 16)
        def _(blk):
            cb = wid * _NT + blk * 16
            pltpu.sync_copy(eidx_hbm.at[0, pl.ds(cb, 16)], idx_all.at[0])
            pltpu.sync_copy(eidx_hbm.at[1, pl.ds(cb, 16)], idx_all.at[1])

            @pl.loop(0, 16, step=2)
            def _(u):
                gh = [
                    pltpu.async_copy(
                        xs_hbm.at[idx_all.at[0, u + b]], rows_v.at[b], gsems[b]
                    )
                    for b in range(2)
                ]
                sh = []
                for b in range(2):
                    gh[b].wait()
                    sh.append(
                        pltpu.async_copy(
                            rows_v.at[b], acc.at[idx_all.at[1, u + b]],
                            ssems[b], add=True,
                        )
                    )
                for b in range(2):
                    sh[b].wait()

        plsc.subcore_barrier()
        pltpu.sync_copy(acc.at[pl.ds(base, _RPT)], out_hbm.at[c, pl.ds(base, _RPT)])

    return k(xs, eidx, jnp.zeros((_NPAD, feat), jnp.float32))


def _degcount(eidx):
    @functools.partial(
        pl.kernel,
        out_type=jax.ShapeDtypeStruct((_NC, _NPAD, 128), jnp.float32),
        mesh=_sc_mesh(),
        scratch_types=[
            pltpu.VMEM((_NT, _CHUNK), jnp.int32),
            pltpu.VMEM((_CHUNK, 128), jnp.float32),
            pltpu.VMEM_SHARED((_NPAD, 128), jnp.float32),
        ],
    )
    def k(eidx_hbm, ones_hbm, zeros_hbm, out_hbm, idx_all, ones_v, acc):
        c = lax.axis_index("c")
        s = lax.axis_index("s")
        wid = s * _NC + c
        base = s * _RPT
        pltpu.sync_copy(eidx_hbm.at[0, pl.ds(wid * _NT, _NT)], idx_all)
        pltpu.sync_copy(ones_hbm, ones_v)
        pltpu.sync_copy(zeros_hbm.at[pl.ds(base, _RPT)], acc.at[pl.ds(base, _RPT)])
        plsc.subcore_barrier()

        @pl.loop(0, _NT)
        def _(t):
            pltpu.sync_copy(ones_v, acc.at[idx_all.at[t]], add=True)

        plsc.subcore_barrier()
        pltpu.sync_copy(acc.at[pl.ds(base, _RPT)], out_hbm.at[c, pl.ds(base, _RPT)])

    return k(
        eidx,
        jnp.ones((_CHUNK, 128), jnp.float32),
        jnp.zeros((_NPAD, 128), jnp.float32),
    )


def _bn_lrelu(h, g, be):
    m = jnp.mean(h, axis=0, keepdims=True)
    v = jnp.mean(h * h, axis=0, keepdims=True) - m * m
    hn = g * (h - m) * lax.rsqrt(v + 1e-5) + be
    return jnp.where(hn > 0, hn, 0.01 * hn)


def _t1(x_ref, dg_ref, w10_ref, b1_ref, dinv_ref, xs_ref, p1_ref):
    deg = dg_ref[0, :_N, 0:1] + dg_ref[1, :_N, 0:1]
    dinv = jnp.where(deg > 0, lax.rsqrt(deg), 0.0)
    dinv_ref[...] = dinv
    xv = x_ref[...]
    xs_ref[0:_N, :] = xv * dinv
    xs_ref[_N:_NPAD, :] = jnp.zeros((_NPAD - _N, 128), jnp.float32)
    p1_ref[...] = (
        jnp.dot(xv, w10_ref[...], preferred_element_type=jnp.float32) + b1_ref[...]
    )


def _t2(p1_ref, acc_ref, dinv_ref, w11_ref, g1_ref, be1_ref, w20_ref, b2_ref,
        p2_ref, y2_ref):
    dinv = dinv_ref[...]
    tx = -(acc_ref[0, :_N, :] + acc_ref[1, :_N, :]) * dinv
    h = p1_ref[...] + jnp.dot(tx, w11_ref[...], preferred_element_type=jnp.float32)
    h = _bn_lrelu(h, g1_ref[...], be1_ref[...])
    p2_ref[...] = (
        jnp.dot(h, w20_ref[...], preferred_element_type=jnp.float32) + b2_ref[...]
    )
    y2_ref[0:_N, :] = h * dinv
    y2_ref[_N:_NPAD, :] = jnp.zeros((_NPAD - _N, 128), jnp.float32)


def _t3(p2_ref, acc_ref, dinv_ref, w21_ref, g2_ref, be2_ref, w30_ref, b3_ref,
        p3_ref, y3_ref):
    dinv = dinv_ref[...]
    tx = -(acc_ref[0, :_N, :] + acc_ref[1, :_N, :]) * dinv
    h2 = p2_ref[...] + jnp.dot(tx, w21_ref[...], preferred_element_type=jnp.float32)
    z = _bn_lrelu(h2, g2_ref[...], be2_ref[...])
    p3_ref[...] = (
        jnp.dot(z, w30_ref[...], preferred_element_type=jnp.float32) + b3_ref[...]
    )
    y3_ref[0:_N, 0:64] = z * dinv
    y3_ref[0:_N, 64:128] = jnp.zeros((_N, 64), jnp.float32)
    y3_ref[_N:_NPAD, :] = jnp.zeros((_NPAD - _N, 128), jnp.float32)


def _t4(p3_ref, acc_ref, dinv_ref, w31_ref, out_ref):
    tx = -(acc_ref[0, :_N, 0:64] + acc_ref[1, :_N, 0:64]) * dinv_ref[...]
    out_ref[...] = p3_ref[...] + jnp.dot(
        tx, w31_ref[...], preferred_element_type=jnp.float32
    )


def _f32(shape):
    return jax.ShapeDtypeStruct(shape, jnp.float32)


def kernel(x, edge_index, W1, b1, W2, b2, W3, b3, g1, be1, g2, be2):
    C, H, Hh = 128, 128, 64
    padidx = _N + jnp.arange(_EP - _E, dtype=jnp.int32) % (_NPAD - _N)
    epad = jnp.stack([padidx, padidx])
    eidx = jnp.concatenate([edge_index, epad], axis=1).reshape(2, _NCHUNKS, _CHUNK)

    dg = _degcount(eidx)

    dinv, xs, p1 = pl.pallas_call(
        _t1, out_shape=(_f32((_N, 1)), _f32((_NPAD, C)), _f32((_N, H))),
    )(x, dg, W1[0], b1.reshape(1, H))

    acc1 = _segsum(xs, eidx, C)

    p2, y2 = pl.pallas_call(
        _t2, out_shape=(_f32((_N, Hh)), _f32((_NPAD, H))),
    )(p1, acc1, dinv, W1[1], g1.reshape(1, H), be1.reshape(1, H),
      W2[0], b2.reshape(1, Hh))

    acc2 = _segsum(y2, eidx, H)

    p3, y3 = pl.pallas_call(
        _t3, out_shape=(_f32((_N, C)), _f32((_NPAD, H))),
    )(p2, acc2, dinv, W2[1], g2.reshape(1, Hh), be2.reshape(1, Hh),
      W3[0], b3.reshape(1, C))

    acc3 = _segsum(y3, eidx, H)

    out = pl.pallas_call(
        _t4, out_shape=_f32((_N, C)),
    )(p3, acc3, dinv, W3[1])

    return out


# speedup vs baseline: 1.0230x; 1.0230x over previous
import functools

import jax
import jax.numpy as jnp
from jax import lax
from jax.experimental import pallas as pl
from jax.experimental.pallas import tpu as pltpu
from jax.experimental.pallas import tpu_sc as plsc

_N = 10000
_NPAD = 10240
_E = 320000
_CHUNK = 128
_NC = 2
_NS = 16
_NW = _NC * _NS
_NT = 80
_NCHUNKS = _NW * _NT
_EP = _NCHUNKS * _CHUNK
_RPT = _NPAD // _NS


def _sc_mesh():
    return plsc.VectorSubcoreMesh(core_axis_name="c", subcore_axis_name="s")


def _segsum(xs, eidx, feat):
    @functools.partial(
        pl.kernel,
        out_type=jax.ShapeDtypeStruct((_NC, _NPAD, feat), jnp.float32),
        mesh=_sc_mesh(),
        scratch_types=[
            pltpu.VMEM((2, 2, 16, _CHUNK), jnp.int32),
            pltpu.VMEM((2, _CHUNK, feat), jnp.float32),
            pltpu.VMEM_SHARED((_NPAD, feat), jnp.float32),
            pltpu.SemaphoreType.DMA,
            pltpu.SemaphoreType.DMA,
            pltpu.SemaphoreType.DMA,
            pltpu.SemaphoreType.DMA,
            pltpu.SemaphoreType.DMA,
            pltpu.SemaphoreType.DMA,
        ],
    )
    def k(xs_hbm, eidx_hbm, zeros_hbm, out_hbm, idx_all, rows_v, acc,
          g0, g1, s0, s1, i0, i1):
        gsems = (g0, g1)
        ssems = (s0, s1)
        isems = (i0, i1)
        c = lax.axis_index("c")
        s = lax.axis_index("s")
        wid = s * _NC + c
        base = s * _RPT
        pltpu.sync_copy(zeros_hbm.at[pl.ds(base, _RPT)], acc.at[pl.ds(base, _RPT)])
        plsc.subcore_barrier()

        nblk = _NT // 16

        def load_idx(blk):
            cb = wid * _NT + blk * 16
            ib = blk % 2
            return [
                pltpu.async_copy(
                    eidx_hbm.at[r, pl.ds(cb, 16)], idx_all.at[ib, r], isems[ib]
                )
                for r in range(2)
            ]

        ih = {0: load_idx(0)}
        for blk in range(nblk):
            if blk + 1 < nblk:
                ih[blk + 1] = load_idx(blk + 1)
            for h in ih.pop(blk):
                h.wait()
            ib = blk % 2

            @pl.loop(0, 16, step=2)
            def _(u):
                gh = [
                    pltpu.async_copy(
                        xs_hbm.at[idx_all.at[ib, 0, u + b]], rows_v.at[b],
                        gsems[b],
                    )
                    for b in range(2)
                ]
                sh = []
                for b in range(2):
                    gh[b].wait()
                    sh.append(
                        pltpu.async_copy(
                            rows_v.at[b], acc.at[idx_all.at[ib, 1, u + b]],
                            ssems[b], add=True,
                        )
                    )
                for b in range(2):
                    sh[b].wait()

        plsc.subcore_barrier()
        pltpu.sync_copy(acc.at[pl.ds(base, _RPT)], out_hbm.at[c, pl.ds(base, _RPT)])

    return k(xs, eidx, jnp.zeros((_NPAD, feat), jnp.float32))


def _degcount(eidx):
    @functools.partial(
        pl.kernel,
        out_type=jax.ShapeDtypeStruct((_NC, _NPAD, 128), jnp.float32),
        mesh=_sc_mesh(),
        scratch_types=[
            pltpu.VMEM((_NT, _CHUNK), jnp.int32),
            pltpu.VMEM((_CHUNK, 128), jnp.float32),
            pltpu.VMEM_SHARED((_NPAD, 128), jnp.float32),
            pltpu.SemaphoreType.DMA,
            pltpu.SemaphoreType.DMA,
        ],
    )
    def k(eidx_hbm, ones_hbm, zeros_hbm, out_hbm, idx_all, ones_v, acc, d0, d1):
        dsems = (d0, d1)
        c = lax.axis_index("c")
        s = lax.axis_index("s")
        wid = s * _NC + c
        base = s * _RPT
        pltpu.sync_copy(eidx_hbm.at[0, pl.ds(wid * _NT, _NT)], idx_all)
        pltpu.sync_copy(ones_hbm, ones_v)
        pltpu.sync_copy(zeros_hbm.at[pl.ds(base, _RPT)], acc.at[pl.ds(base, _RPT)])
        plsc.subcore_barrier()

        @pl.loop(0, _NT, step=2)
        def _(t):
            dh = [
                pltpu.async_copy(
                    ones_v, acc.at[idx_all.at[t + b]], dsems[b], add=True
                )
                for b in range(2)
            ]
            for b in range(2):
                dh[b].wait()

        plsc.subcore_barrier()
        pltpu.sync_copy(acc.at[pl.ds(base, _RPT)], out_hbm.at[c, pl.ds(base, _RPT)])

    return k(
        eidx,
        jnp.ones((_CHUNK, 128), jnp.float32),
        jnp.zeros((_NPAD, 128), jnp.float32),
    )


def _bn_lrelu(h, g, be):
    m = jnp.mean(h, axis=0, keepdims=True)
    v = jnp.mean(h * h, axis=0, keepdims=True) - m * m
    hn = g * (h - m) * lax.rsqrt(v + 1e-5) + be
    return jnp.where(hn > 0, hn, 0.01 * hn)


def _t1(x_ref, dg_ref, w10_ref, b1_ref, dinv_ref, xs_ref, p1_ref):
    deg = dg_ref[0, :_N, 0:1] + dg_ref[1, :_N, 0:1]
    dinv = jnp.where(deg > 0, lax.rsqrt(deg), 0.0)
    dinv_ref[...] = dinv
    xv = x_ref[...]
    xs_ref[0:_N, :] = xv * dinv
    xs_ref[_N:_NPAD, :] = jnp.zeros((_NPAD - _N, 128), jnp.float32)
    p1_ref[...] = (
        jnp.dot(xv, w10_ref[...], preferred_element_type=jnp.float32) + b1_ref[...]
    )


def _t2(p1_ref, acc_ref, dinv_ref, w11_ref, g1_ref, be1_ref, w20_ref, b2_ref,
        p2_ref, y2_ref):
    dinv = dinv_ref[...]
    tx = -(acc_ref[0, :_N, :] + acc_ref[1, :_N, :]) * dinv
    h = p1_ref[...] + jnp.dot(tx, w11_ref[...], preferred_element_type=jnp.float32)
    h = _bn_lrelu(h, g1_ref[...], be1_ref[...])
    p2_ref[...] = (
        jnp.dot(h, w20_ref[...], preferred_element_type=jnp.float32) + b2_ref[...]
    )
    y2_ref[0:_N, :] = h * dinv
    y2_ref[_N:_NPAD, :] = jnp.zeros((_NPAD - _N, 128), jnp.float32)


def _t3(p2_ref, acc_ref, dinv_ref, w21_ref, g2_ref, be2_ref, w30_ref, b3_ref,
        p3_ref, y3_ref):
    dinv = dinv_ref[...]
    tx = -(acc_ref[0, :_N, :] + acc_ref[1, :_N, :]) * dinv
    h2 = p2_ref[...] + jnp.dot(tx, w21_ref[...], preferred_element_type=jnp.float32)
    z = _bn_lrelu(h2, g2_ref[...], be2_ref[...])
    p3_ref[...] = (
        jnp.dot(z, w30_ref[...], preferred_element_type=jnp.float32) + b3_ref[...]
    )
    y3_ref[0:_N, 0:64] = z * dinv
    y3_ref[0:_N, 64:128] = jnp.zeros((_N, 64), jnp.float32)
    y3_ref[_N:_NPAD, :] = jnp.zeros((_NPAD - _N, 128), jnp.float32)


def _t4(p3_ref, acc_ref, dinv_ref, w31_ref, out_ref):
    tx = -(acc_ref[0, :_N, 0:64] + acc_ref[1, :_N, 0:64]) * dinv_ref[...]
    out_ref[...] = p3_ref[...] + jnp.dot(
        tx, w31_ref[...], preferred_element_type=jnp.float32
    )


def _f32(shape):
    return jax.ShapeDtypeStruct(shape, jnp.float32)


def kernel(x, edge_index, W1, b1, W2, b2, W3, b3, g1, be1, g2, be2):
    C, H, Hh = 128, 128, 64
    padidx = _N + jnp.arange(_EP - _E, dtype=jnp.int32) % (_NPAD - _N)
    epad = jnp.stack([padidx, padidx])
    eidx = jnp.concatenate([edge_index, epad], axis=1).reshape(2, _NCHUNKS, _CHUNK)

    dg = _degcount(eidx)

    dinv, xs, p1 = pl.pallas_call(
        _t1, out_shape=(_f32((_N, 1)), _f32((_NPAD, C)), _f32((_N, H))),
    )(x, dg, W1[0], b1.reshape(1, H))

    acc1 = _segsum(xs, eidx, C)

    p2, y2 = pl.pallas_call(
        _t2, out_shape=(_f32((_N, Hh)), _f32((_NPAD, H))),
    )(p1, acc1, dinv, W1[1], g1.reshape(1, H), be1.reshape(1, H),
      W2[0], b2.reshape(1, Hh))

    acc2 = _segsum(y2, eidx, H)

    p3, y3 = pl.pallas_call(
        _t3, out_shape=(_f32((_N, C)), _f32((_NPAD, H))),
    )(p2, acc2, dinv, W2[1], g2.reshape(1, Hh), be2.reshape(1, Hh),
      W3[0], b3.reshape(1, C))

    acc3 = _segsum(y3, eidx, H)

    out = pl.pallas_call(
        _t4, out_shape=_f32((_N, C)),
    )(p3, acc3, dinv, W3[1])

    return out


# speedup vs baseline: 1.0418x; 1.0183x over previous
import functools

import jax
import jax.numpy as jnp
from jax import lax
from jax.experimental import pallas as pl
from jax.experimental.pallas import tpu as pltpu
from jax.experimental.pallas import tpu_sc as plsc

_N = 10000
_NPAD = 10240
_E = 320000
_CHUNK = 128
_NC = 2
_NS = 16
_NW = _NC * _NS
_NT = 80
_NCHUNKS = _NW * _NT
_EP = _NCHUNKS * _CHUNK
_RPT = _NPAD // _NS


def _sc_mesh():
    return plsc.VectorSubcoreMesh(core_axis_name="c", subcore_axis_name="s")


def _segsum(xs, eidx, feat):
    @functools.partial(
        pl.kernel,
        out_type=jax.ShapeDtypeStruct((_NC, _NPAD, feat), jnp.float32),
        mesh=_sc_mesh(),
        scratch_types=[
            pltpu.VMEM((2, 2, 16, _CHUNK), jnp.int32),
            pltpu.VMEM((2, _CHUNK, feat), jnp.float32),
            pltpu.VMEM_SHARED((_NPAD, feat), jnp.float32),
            pltpu.SemaphoreType.DMA,
            pltpu.SemaphoreType.DMA,
            pltpu.SemaphoreType.DMA,
            pltpu.SemaphoreType.DMA,
            pltpu.SemaphoreType.DMA,
            pltpu.SemaphoreType.DMA,
        ],
    )
    def k(xs_hbm, eidx_hbm, out_hbm, idx_all, rows_v, acc,
          g0, g1, s0, s1, i0, i1):
        gsems = (g0, g1)
        ssems = (s0, s1)
        isems = (i0, i1)
        c = lax.axis_index("c")
        s = lax.axis_index("s")
        wid = s * _NC + c
        base = s * _RPT

        @pl.loop(0, _CHUNK)
        def _(i):
            for j in range(0, feat, 16):
                rows_v[0, i, j:j + 16] = jnp.zeros((16,), jnp.float32)

        for r in range(_RPT // _CHUNK):
            pltpu.sync_copy(
                rows_v.at[0], acc.at[pl.ds(base + r * _CHUNK, _CHUNK)]
            )
        plsc.subcore_barrier()

        nblk = _NT // 16

        def load_idx(blk):
            cb = wid * _NT + blk * 16
            ib = blk % 2
            return [
                pltpu.async_copy(
                    eidx_hbm.at[r, pl.ds(cb, 16)], idx_all.at[ib, r], isems[ib]
                )
                for r in range(2)
            ]

        ih = {0: load_idx(0)}
        for blk in range(nblk):
            if blk + 1 < nblk:
                ih[blk + 1] = load_idx(blk + 1)
            for h in ih.pop(blk):
                h.wait()
            ib = blk % 2

            @pl.loop(0, 16, step=2)
            def _(u):
                gh = [
                    pltpu.async_copy(
                        xs_hbm.at[idx_all.at[ib, 0, u + b]], rows_v.at[b],
                        gsems[b],
                    )
                    for b in range(2)
                ]
                sh = []
                for b in range(2):
                    gh[b].wait()
                    sh.append(
                        pltpu.async_copy(
                            rows_v.at[b], acc.at[idx_all.at[ib, 1, u + b]],
                            ssems[b], add=True,
                        )
                    )
                for b in range(2):
                    sh[b].wait()

        plsc.subcore_barrier()
        pltpu.sync_copy(acc.at[pl.ds(base, _RPT)], out_hbm.at[c, pl.ds(base, _RPT)])

    return k(xs, eidx)


def _degcount(eidx):
    @functools.partial(
        pl.kernel,
        out_type=jax.ShapeDtypeStruct((_NC, _NPAD, 128), jnp.float32),
        mesh=_sc_mesh(),
        scratch_types=[
            pltpu.VMEM((_NT, _CHUNK), jnp.int32),
            pltpu.VMEM((_CHUNK, 128), jnp.float32),
            pltpu.VMEM_SHARED((_NPAD, 128), jnp.float32),
            pltpu.SemaphoreType.DMA,
            pltpu.SemaphoreType.DMA,
        ],
    )
    def k(eidx_hbm, ones_hbm, zeros_hbm, out_hbm, idx_all, ones_v, acc, d0, d1):
        dsems = (d0, d1)
        c = lax.axis_index("c")
        s = lax.axis_index("s")
        wid = s * _NC + c
        base = s * _RPT
        pltpu.sync_copy(eidx_hbm.at[0, pl.ds(wid * _NT, _NT)], idx_all)
        pltpu.sync_copy(ones_hbm, ones_v)
        pltpu.sync_copy(zeros_hbm.at[pl.ds(base, _RPT)], acc.at[pl.ds(base, _RPT)])
        plsc.subcore_barrier()

        @pl.loop(0, _NT, step=2)
        def _(t):
            dh = [
                pltpu.async_copy(
                    ones_v, acc.at[idx_all.at[t + b]], dsems[b], add=True
                )
                for b in range(2)
            ]
            for b in range(2):
                dh[b].wait()

        plsc.subcore_barrier()
        pltpu.sync_copy(acc.at[pl.ds(base, _RPT)], out_hbm.at[c, pl.ds(base, _RPT)])

    return k(
        eidx,
        jnp.ones((_CHUNK, 128), jnp.float32),
        jnp.zeros((_NPAD, 128), jnp.float32),
    )


def _bn_lrelu(h, g, be):
    m = jnp.mean(h, axis=0, keepdims=True)
    v = jnp.mean(h * h, axis=0, keepdims=True) - m * m
    hn = g * (h - m) * lax.rsqrt(v + 1e-5) + be
    return jnp.where(hn > 0, hn, 0.01 * hn)


def _t1(x_ref, dg_ref, w10_ref, b1_ref, dinv_ref, xs_ref, p1_ref):
    deg = dg_ref[0, :_N, 0:1] + dg_ref[1, :_N, 0:1]
    dinv = jnp.where(deg > 0, lax.rsqrt(deg), 0.0)
    dinv_ref[...] = dinv
    xv = x_ref[...]
    xs_ref[0:_N, :] = xv * dinv
    xs_ref[_N:_NPAD, :] = jnp.zeros((_NPAD - _N, 128), jnp.float32)
    p1_ref[...] = (
        jnp.dot(xv, w10_ref[...], preferred_element_type=jnp.float32) + b1_ref[...]
    )


def _t2(p1_ref, acc_ref, dinv_ref, w11_ref, g1_ref, be1_ref, w20_ref, b2_ref,
        p2_ref, y2_ref):
    dinv = dinv_ref[...]
    tx = -(acc_ref[0, :_N, :] + acc_ref[1, :_N, :]) * dinv
    h = p1_ref[...] + jnp.dot(tx, w11_ref[...], preferred_element_type=jnp.float32)
    h = _bn_lrelu(h, g1_ref[...], be1_ref[...])
    p2_ref[...] = (
        jnp.dot(h, w20_ref[...], preferred_element_type=jnp.float32) + b2_ref[...]
    )
    y2_ref[0:_N, :] = h * dinv
    y2_ref[_N:_NPAD, :] = jnp.zeros((_NPAD - _N, 128), jnp.float32)


def _t3(p2_ref, acc_ref, dinv_ref, w21_ref, g2_ref, be2_ref, w30_ref, b3_ref,
        p3_ref, y3_ref):
    dinv = dinv_ref[...]
    tx = -(acc_ref[0, :_N, :] + acc_ref[1, :_N, :]) * dinv
    h2 = p2_ref[...] + jnp.dot(tx, w21_ref[...], preferred_element_type=jnp.float32)
    z = _bn_lrelu(h2, g2_ref[...], be2_ref[...])
    p3_ref[...] = (
        jnp.dot(z, w30_ref[...], preferred_element_type=jnp.float32) + b3_ref[...]
    )
    y3_ref[0:_N, 0:64] = z * dinv
    y3_ref[0:_N, 64:128] = jnp.zeros((_N, 64), jnp.float32)
    y3_ref[_N:_NPAD, :] = jnp.zeros((_NPAD - _N, 128), jnp.float32)


def _t4(p3_ref, acc_ref, dinv_ref, w31_ref, out_ref):
    tx = -(acc_ref[0, :_N, 0:64] + acc_ref[1, :_N, 0:64]) * dinv_ref[...]
    out_ref[...] = p3_ref[...] + jnp.dot(
        tx, w31_ref[...], preferred_element_type=jnp.float32
    )


def _f32(shape):
    return jax.ShapeDtypeStruct(shape, jnp.float32)


def kernel(x, edge_index, W1, b1, W2, b2, W3, b3, g1, be1, g2, be2):
    C, H, Hh = 128, 128, 64
    padidx = _N + jnp.arange(_EP - _E, dtype=jnp.int32) % (_NPAD - _N)
    epad = jnp.stack([padidx, padidx])
    eidx = jnp.concatenate([edge_index, epad], axis=1).reshape(2, _NCHUNKS, _CHUNK)

    dg = _degcount(eidx)

    dinv, xs, p1 = pl.pallas_call(
        _t1, out_shape=(_f32((_N, 1)), _f32((_NPAD, C)), _f32((_N, H))),
    )(x, dg, W1[0], b1.reshape(1, H))

    acc1 = _segsum(xs, eidx, C)

    p2, y2 = pl.pallas_call(
        _t2, out_shape=(_f32((_N, Hh)), _f32((_NPAD, H))),
    )(p1, acc1, dinv, W1[1], g1.reshape(1, H), be1.reshape(1, H),
      W2[0], b2.reshape(1, Hh))

    acc2 = _segsum(y2, eidx, H)

    p3, y3 = pl.pallas_call(
        _t3, out_shape=(_f32((_N, C)), _f32((_NPAD, H))),
    )(p2, acc2, dinv, W2[1], g2.reshape(1, Hh), be2.reshape(1, Hh),
      W3[0], b3.reshape(1, C))

    acc3 = _segsum(y3, eidx, H)

    out = pl.pallas_call(
        _t4, out_shape=_f32((_N, C)),
    )(p3, acc3, dinv, W3[1])

    return out
